# baseline (speedup 1.0000x reference)
import jax
import jax.numpy as jnp
from jax import lax
from jax.experimental import pallas as pl
from jax.experimental.pallas import tpu as pltpu
from jax.experimental.pallas import tpu_sc as plsc

N_USER = 50000
N_ITEM = 50000
N = N_USER + N_ITEM
EMB = 32
HEMB = 16
NNZ = 1600000

ACC_ROWS = 100352
TILE_SPAN = 6272

N_TILES = 16
BLK = 512
CHUNKS = 4
CHUNK = 128
BLOCKS_PER_TILE = 196
EDGES_PER_TILE = BLOCKS_PER_TILE * BLK
NNZ_PAD = N_TILES * EDGES_PER_TILE
ROW_BLK = 4000


def _sc_body(ego_hbm, rows_hbm, cols_hbm, vals_hbm, out_hbm,
             acc, rowFA, colFA, valA, msgA, sidxA,
             rowFB, colFB, valB, msgB, sidxB,
             esemA, esemB, gsemA, gsemB, ssemA, ssemB):
    cid = lax.axis_index("c")
    sid = lax.axis_index("s")
    coff = cid * N

    zero16 = jnp.zeros((16,), jnp.float32)
    izero16 = jnp.zeros((16,), jnp.int32)

    def _edge_start(b, rowFb, colFb, valb, esem):
        off = pl.multiple_of((sid * BLOCKS_PER_TILE + b) * BLK, BLK)
        pltpu.async_copy(rows_hbm.at[pl.ds(off, BLK)], rowFb, esem)
        pltpu.async_copy(cols_hbm.at[pl.ds(off, BLK)], colFb, esem)
        pltpu.async_copy(vals_hbm.at[pl.ds(off, BLK)], valb, esem)

    def _edge_wait(rowFb, colFb, valb, esem):
        pltpu.make_async_copy(rows_hbm.at[pl.ds(0, BLK)], rowFb, esem).wait()
        pltpu.make_async_copy(cols_hbm.at[pl.ds(0, BLK)], colFb, esem).wait()
        pltpu.make_async_copy(vals_hbm.at[pl.ds(0, BLK)], valb, esem).wait()

    def _prep(rowFb, colFb, sidxb):
        for j in range(CHUNKS):
            for g in range(8):
                sl = pl.ds((j * 8 + g) * 16, 16)
                sidxb[j, pl.ds(g * 16, 16)] = rowFb[sl]
                colFb[sl] = colFb[sl] + coff

    def _gather_start(colFb, msgb, gsem):
        for j in range(CHUNKS):
            pltpu.async_copy(ego_hbm.at[colFb.at[pl.ds(j * CHUNK, CHUNK)]],
                             msgb.at[pl.ds(j * CHUNK, CHUNK)], gsem)

    def _gather_wait(msgb, gsem):
        pltpu.make_async_copy(ego_hbm.at[pl.ds(0, BLK)], msgb, gsem).wait()

    def _scale(valb, msgb):
        def body(i, _):
            vv = valb[pl.ds(i * 16, 16)]
            for k in range(16):
                e = i * 16 + k
                msgb[e, 0:16] = msgb[e, 0:16] * vv[k]
            return 0
        lax.fori_loop(0, BLK // 16, body, 0)

    def _scatter_start(sidxb, msgb, ssem):
        for j in range(CHUNKS):
            pltpu.async_copy(msgb.at[pl.ds(j * CHUNK, CHUNK)],
                             acc.at[sidxb.at[j]], ssem, add=True)

    def _scatter_wait(msgb, ssem):
        pltpu.make_async_copy(ego_hbm.at[pl.ds(0, BLK)], msgb, ssem).wait()

    def _zmsg(i, _):
        msgA[i, 0:16] = zero16
        msgB[i, 0:16] = zero16
        return 0
    lax.fori_loop(0, BLK, _zmsg, 0)
    for j in range(CHUNKS):
        for g in range(8):
            sidxB[j, pl.ds(g * 16, 16)] = izero16
    zbase = sid * TILE_SPAN
    for k in range(12):
        pltpu.async_copy(msgA.at[pl.ds(0, BLK)],
                         acc.at[pl.ds(zbase + k * BLK, BLK)], gsemA)
    pltpu.async_copy(msgA.at[pl.ds(0, 128)],
                     acc.at[pl.ds(zbase + 12 * BLK, 128)], gsemA)
    for k in range(12):
        pltpu.make_async_copy(msgA.at[pl.ds(0, BLK)],
                              acc.at[pl.ds(zbase + k * BLK, BLK)], gsemA).wait()
    pltpu.make_async_copy(msgA.at[pl.ds(0, 128)],
                          acc.at[pl.ds(zbase + 12 * BLK, 128)], gsemA).wait()
    plsc.subcore_barrier()

    _scatter_start(sidxB, msgB, ssemB)
    _edge_start(0, rowFA, colFA, valA, esemA)
    _edge_start(1, rowFB, colFB, valB, esemB)
    _edge_wait(rowFA, colFA, valA, esemA)
    _prep(rowFA, colFA, sidxA)
    _gather_start(colFA, msgA, gsemA)

    def _iter(i, _):
        a = 2 * i
        b = 2 * i + 1
        _gather_wait(msgA, gsemA)
        _scale(valA, msgA)
        _scatter_start(sidxA, msgA, ssemA)
        _edge_start(a + 2, rowFA, colFA, valA, esemA)
        _edge_wait(rowFB, colFB, valB, esemB)
        _scatter_wait(msgB, ssemB)
        _prep(rowFB, colFB, sidxB)
        _gather_start(colFB, msgB, gsemB)
        _gather_wait(msgB, gsemB)
        _scale(valB, msgB)
        _scatter_start(sidxB, msgB, ssemB)
        _edge_start(b + 2, rowFB, colFB, valB, esemB)
        _edge_wait(rowFA, colFA, valA, esemA)
        _scatter_wait(msgA, ssemA)
        _prep(rowFA, colFA, sidxA)
        _gather_start(colFA, msgA, gsemA)
        return 0

    lax.fori_loop(0, BLOCKS_PER_TILE // 2 - 1, _iter, 0)

    _gather_wait(msgA, gsemA)
    _scale(valA, msgA)
    _scatter_start(sidxA, msgA, ssemA)
    _edge_wait(rowFB, colFB, valB, esemB)
    _scatter_wait(msgB, ssemB)
    _prep(rowFB, colFB, sidxB)
    _gather_start(colFB, msgB, gsemB)
    _gather_wait(msgB, gsemB)
    _scale(valB, msgB)
    _scatter_start(sidxB, msgB, ssemB)
    _scatter_wait(msgA, ssemA)
    _scatter_wait(msgB, ssemB)
    plsc.subcore_barrier()

    @pl.when(sid < N_TILES - 1)
    def _():
        pltpu.sync_copy(acc.at[pl.ds(sid * TILE_SPAN, TILE_SPAN)],
                        out_hbm.at[pl.ds(cid * N + sid * TILE_SPAN, TILE_SPAN)])

    @pl.when(sid == N_TILES - 1)
    def _():
        pltpu.sync_copy(acc.at[pl.ds((N_TILES - 1) * TILE_SPAN, 5920)],
                        out_hbm.at[pl.ds(cid * N + (N_TILES - 1) * TILE_SPAN,
                                         5920)])


@jax.jit
def _sc_spmm(ego_split, rows, cols, vals):
    return pl.kernel(
        _sc_body,
        out_type=jax.ShapeDtypeStruct((2 * N, HEMB), jnp.float32),
        mesh=plsc.VectorSubcoreMesh(core_axis_name="c", subcore_axis_name="s"),
        compiler_params=pltpu.CompilerParams(use_tc_tiling_on_sc=False),
        scratch_types=[
            pltpu.VMEM_SHARED((ACC_ROWS, HEMB), jnp.float32),
            pltpu.VMEM((BLK,), jnp.int32),
            pltpu.VMEM((BLK,), jnp.int32),
            pltpu.VMEM((BLK,), jnp.float32),
            pltpu.VMEM((BLK, HEMB), jnp.float32),
            pltpu.VMEM((CHUNKS, CHUNK), jnp.int32),
            pltpu.VMEM((BLK,), jnp.int32),
            pltpu.VMEM((BLK,), jnp.int32),
            pltpu.VMEM((BLK,), jnp.float32),
            pltpu.VMEM((BLK, HEMB), jnp.float32),
            pltpu.VMEM((CHUNKS, CHUNK), jnp.int32),
            pltpu.SemaphoreType.DMA,
            pltpu.SemaphoreType.DMA,
            pltpu.SemaphoreType.DMA,
            pltpu.SemaphoreType.DMA,
            pltpu.SemaphoreType.DMA,
            pltpu.SemaphoreType.DMA,
        ],
    )(ego_split, rows, cols, vals)


def _dense_body(le_lo, le_hi, ego_lo, ego_hi, w1_ref, b1_ref, w2_ref, b2_ref,
                ego_out_ref, norm_out_ref):
    c = pl.program_id(1)
    le = jnp.concatenate([le_lo[...], le_hi[...]], axis=1)
    ego = jnp.concatenate([ego_lo[...], ego_hi[...]], axis=1)
    x1 = (le + ego) @ w1_ref[...] + b1_ref[...]
    x2 = (le * ego) @ w2_ref[...] + b2_ref[...]
    m = x1 + x2
    act = jnp.where(m >= 0, m, 0.2 * m)
    n = jnp.sqrt(jnp.sum(act * act, axis=1, keepdims=True))
    norm = act / jnp.maximum(n, 1e-12)
    ego_out_ref[...] = jnp.where(c == 0, act[:, :HEMB], act[:, HEMB:])
    norm_out_ref[...] = norm


@jax.jit
def _dense_layer(le_split, ego_split, w1, b1, w2, b2):
    grid = (N // ROW_BLK, 2)
    return pl.pallas_call(
        _dense_body,
        grid=grid,
        in_specs=[
            pl.BlockSpec((ROW_BLK, HEMB), lambda i, c: (i, 0)),
            pl.BlockSpec((ROW_BLK, HEMB), lambda i, c: (N // ROW_BLK + i, 0)),
            pl.BlockSpec((ROW_BLK, HEMB), lambda i, c: (i, 0)),
            pl.BlockSpec((ROW_BLK, HEMB), lambda i, c: (N // ROW_BLK + i, 0)),
            pl.BlockSpec((EMB, EMB), lambda i, c: (0, 0)),
            pl.BlockSpec((EMB,), lambda i, c: (0,)),
            pl.BlockSpec((EMB, EMB), lambda i, c: (0, 0)),
            pl.BlockSpec((EMB,), lambda i, c: (0,)),
        ],
        out_specs=[
            pl.BlockSpec((ROW_BLK, HEMB),
                         lambda i, c: (c * (N // ROW_BLK) + i, 0)),
            pl.BlockSpec((ROW_BLK, EMB), lambda i, c: (i, 0)),
        ],
        out_shape=[
            jax.ShapeDtypeStruct((2 * N, HEMB), jnp.float32),
            jax.ShapeDtypeStruct((N, EMB), jnp.float32),
        ],
    )(le_split, le_split, ego_split, ego_split, w1, b1, w2, b2)


def kernel(users, pos_items, neg_items, user_emb, item_emb, lap_idx, lap_val,
           W1_0, b1_0, W2_0, b2_0, W1_1, b1_1, W2_1, b2_1):
    ego_split = jnp.concatenate(
        [user_emb[:, :HEMB], item_emb[:, :HEMB],
         user_emb[:, HEMB:], item_emb[:, HEMB:]], axis=0)
    pad = NNZ_PAD - NNZ
    rows = jnp.concatenate([lap_idx[0], jnp.zeros((pad,), jnp.int32)])
    cols = jnp.concatenate([lap_idx[1], jnp.zeros((pad,), jnp.int32)])
    vals = jnp.concatenate([lap_val, jnp.zeros((pad,), jnp.float32)])
    norms = []
    for (w1, b1, w2, b2) in ((W1_0, b1_0, W2_0, b2_0), (W1_1, b1_1, W2_1, b2_1)):
        le_split = _sc_spmm(ego_split, rows, cols, vals)
        ego_split, norm = _dense_layer(le_split, ego_split, w1, b1, w2, b2)
        norms.append(norm)
    iu = users - 1
    ip = pos_items - 1
    im = neg_items - 1
    u_out = jnp.concatenate(
        [jnp.take(user_emb, iu, axis=0, mode='wrap')] +
        [jnp.take(nm[:N_USER], iu, axis=0, mode='wrap') for nm in norms], axis=1)
    p_out = jnp.concatenate(
        [jnp.take(item_emb, ip, axis=0, mode='wrap')] +
        [jnp.take(nm[N_USER:], ip, axis=0, mode='wrap') for nm in norms], axis=1)
    n_out = jnp.concatenate(
        [jnp.take(item_emb, im, axis=0, mode='wrap')] +
        [jnp.take(nm[N_USER:], im, axis=0, mode='wrap') for nm in norms], axis=1)
    return (u_out, p_out, n_out)


# speedup vs baseline: 1.0499x; 1.0499x over previous
import jax
import jax.numpy as jnp
from jax import lax
from jax.experimental import pallas as pl
from jax.experimental.pallas import tpu as pltpu
from jax.experimental.pallas import tpu_sc as plsc

N_USER = 50000
N_ITEM = 50000
N = N_USER + N_ITEM
EMB = 32
HEMB = 16
NNZ = 1600000

ACC_ROWS = 100352
TILE_SPAN = 6272

N_TILES = 16
BLK = 512
CHUNKS = 4
CHUNK = 128
BLOCKS_PER_TILE = 196
EDGES_PER_TILE = BLOCKS_PER_TILE * BLK
NNZ_PAD = N_TILES * EDGES_PER_TILE
ROW_BLK = 4000


def _sc_body(ego_hbm, edges_hbm, vals_hbm, out_hbm,
             acc, rowFA, colFA, valA, msgA, sidxA,
             rowFB, colFB, valB, msgB, sidxB,
             esemA, esemB, gsemA, gsemB, ssemA, ssemB):
    cid = lax.axis_index("c")
    sid = lax.axis_index("s")
    coff = cid * N

    zero16 = jnp.zeros((16,), jnp.float32)
    izero16 = jnp.zeros((16,), jnp.int32)

    def _edge_start(b, rowFb, colFb, valb, esem):
        off = pl.multiple_of((sid * BLOCKS_PER_TILE + b) * BLK, BLK)
        pltpu.async_copy(edges_hbm.at[0, pl.ds(off, BLK)], rowFb, esem)
        pltpu.async_copy(edges_hbm.at[1, pl.ds(off, BLK)], colFb, esem)
        pltpu.async_copy(vals_hbm.at[pl.ds(off, BLK)], valb, esem)

    def _edge_wait(rowFb, colFb, valb, esem):
        pltpu.make_async_copy(edges_hbm.at[0, pl.ds(0, BLK)], rowFb, esem).wait()
        pltpu.make_async_copy(edges_hbm.at[1, pl.ds(0, BLK)], colFb, esem).wait()
        pltpu.make_async_copy(vals_hbm.at[pl.ds(0, BLK)], valb, esem).wait()

    def _prep(rowFb, colFb, sidxb):
        for j in range(CHUNKS):
            for g in range(8):
                sl = pl.ds((j * 8 + g) * 16, 16)
                sidxb[j, pl.ds(g * 16, 16)] = rowFb[sl]
                colFb[sl] = colFb[sl] + coff

    def _gather_start(colFb, msgb, gsem):
        for j in range(CHUNKS):
            pltpu.async_copy(ego_hbm.at[colFb.at[pl.ds(j * CHUNK, CHUNK)]],
                             msgb.at[pl.ds(j * CHUNK, CHUNK)], gsem)

    def _gather_wait(msgb, gsem):
        pltpu.make_async_copy(ego_hbm.at[pl.ds(0, BLK)], msgb, gsem).wait()

    def _scale(valb, msgb):
        def body(i, _):
            vv = valb[pl.ds(i * 16, 16)]
            for k in range(16):
                e = i * 16 + k
                msgb[e, 0:16] = msgb[e, 0:16] * vv[k]
            return 0
        lax.fori_loop(0, BLK // 16, body, 0)

    def _scatter_start(sidxb, msgb, ssem):
        for j in range(CHUNKS):
            pltpu.async_copy(msgb.at[pl.ds(j * CHUNK, CHUNK)],
                             acc.at[sidxb.at[j]], ssem, add=True)

    def _scatter_wait(msgb, ssem):
        pltpu.make_async_copy(ego_hbm.at[pl.ds(0, BLK)], msgb, ssem).wait()

    def _zmsg(i, _):
        msgA[i, 0:16] = zero16
        msgB[i, 0:16] = zero16
        return 0
    lax.fori_loop(0, BLK, _zmsg, 0)
    for j in range(CHUNKS):
        for g in range(8):
            sidxB[j, pl.ds(g * 16, 16)] = izero16
    zbase = sid * TILE_SPAN
    for k in range(12):
        pltpu.sync_copy(msgA.at[pl.ds(0, BLK)], acc.at[pl.ds(zbase + k * BLK, BLK)])
    pltpu.sync_copy(msgA.at[pl.ds(0, 128)], acc.at[pl.ds(zbase + 12 * BLK, 128)])
    plsc.subcore_barrier()

    _scatter_start(sidxB, msgB, ssemB)
    _edge_start(0, rowFA, colFA, valA, esemA)
    _edge_start(1, rowFB, colFB, valB, esemB)
    _edge_wait(rowFA, colFA, valA, esemA)
    _prep(rowFA, colFA, sidxA)
    _gather_start(colFA, msgA, gsemA)

    def _iter(i, _):
        a = 2 * i
        b = 2 * i + 1
        _gather_wait(msgA, gsemA)
        _scale(valA, msgA)
        _scatter_start(sidxA, msgA, ssemA)
        _edge_start(a + 2, rowFA, colFA, valA, esemA)
        _edge_wait(rowFB, colFB, valB, esemB)
        _scatter_wait(msgB, ssemB)
        _prep(rowFB, colFB, sidxB)
        _gather_start(colFB, msgB, gsemB)
        _gather_wait(msgB, gsemB)
        _scale(valB, msgB)
        _scatter_start(sidxB, msgB, ssemB)
        _edge_start(b + 2, rowFB, colFB, valB, esemB)
        _edge_wait(rowFA, colFA, valA, esemA)
        _scatter_wait(msgA, ssemA)
        _prep(rowFA, colFA, sidxA)
        _gather_start(colFA, msgA, gsemA)
        return 0

    lax.fori_loop(0, BLOCKS_PER_TILE // 2 - 1, _iter, 0)

    _gather_wait(msgA, gsemA)
    _scale(valA, msgA)
    _scatter_start(sidxA, msgA, ssemA)
    _edge_wait(rowFB, colFB, valB, esemB)
    _scatter_wait(msgB, ssemB)
    _prep(rowFB, colFB, sidxB)
    _gather_start(colFB, msgB, gsemB)
    _gather_wait(msgB, gsemB)
    _scale(valB, msgB)
    _scatter_start(sidxB, msgB, ssemB)
    _scatter_wait(msgA, ssemA)
    _scatter_wait(msgB, ssemB)
    plsc.subcore_barrier()

    @pl.when(sid < N_TILES - 1)
    def _():
        pltpu.sync_copy(acc.at[pl.ds(sid * TILE_SPAN, TILE_SPAN)],
                        out_hbm.at[pl.ds(cid * N + sid * TILE_SPAN, TILE_SPAN)])

    @pl.when(sid == N_TILES - 1)
    def _():
        pltpu.sync_copy(acc.at[pl.ds((N_TILES - 1) * TILE_SPAN, 5920)],
                        out_hbm.at[pl.ds(cid * N + (N_TILES - 1) * TILE_SPAN,
                                         5920)])


@jax.jit
def _sc_spmm(ego_split, edges, vals):
    return pl.kernel(
        _sc_body,
        out_type=jax.ShapeDtypeStruct((2 * N, HEMB), jnp.float32),
        mesh=plsc.VectorSubcoreMesh(core_axis_name="c", subcore_axis_name="s"),
        compiler_params=pltpu.CompilerParams(use_tc_tiling_on_sc=False),
        scratch_types=[
            pltpu.VMEM_SHARED((ACC_ROWS, HEMB), jnp.float32),
            pltpu.VMEM((BLK,), jnp.int32),
            pltpu.VMEM((BLK,), jnp.int32),
            pltpu.VMEM((BLK,), jnp.float32),
            pltpu.VMEM((BLK, HEMB), jnp.float32),
            pltpu.VMEM((CHUNKS, CHUNK), jnp.int32),
            pltpu.VMEM((BLK,), jnp.int32),
            pltpu.VMEM((BLK,), jnp.int32),
            pltpu.VMEM((BLK,), jnp.float32),
            pltpu.VMEM((BLK, HEMB), jnp.float32),
            pltpu.VMEM((CHUNKS, CHUNK), jnp.int32),
            pltpu.SemaphoreType.DMA,
            pltpu.SemaphoreType.DMA,
            pltpu.SemaphoreType.DMA,
            pltpu.SemaphoreType.DMA,
            pltpu.SemaphoreType.DMA,
            pltpu.SemaphoreType.DMA,
        ],
    )(ego_split, edges, vals)


def _dense_body(le_lo, le_hi, ego_lo, ego_hi, w1_ref, b1_ref, w2_ref, b2_ref,
                ego_out_ref, norm_out_ref):
    c = pl.program_id(1)
    le = jnp.concatenate([le_lo[...], le_hi[...]], axis=1)
    ego = jnp.concatenate([ego_lo[...], ego_hi[...]], axis=1)
    x1 = (le + ego) @ w1_ref[...] + b1_ref[...]
    x2 = (le * ego) @ w2_ref[...] + b2_ref[...]
    m = x1 + x2
    act = jnp.where(m >= 0, m, 0.2 * m)
    n = jnp.sqrt(jnp.sum(act * act, axis=1, keepdims=True))
    norm = act / jnp.maximum(n, 1e-12)
    ego_out_ref[...] = jnp.where(c == 0, act[:, :HEMB], act[:, HEMB:])
    norm_out_ref[...] = norm


@jax.jit
def _dense_layer(le_split, ego_split, w1, b1, w2, b2):
    grid = (N // ROW_BLK, 2)
    return pl.pallas_call(
        _dense_body,
        grid=grid,
        in_specs=[
            pl.BlockSpec((ROW_BLK, HEMB), lambda i, c: (i, 0)),
            pl.BlockSpec((ROW_BLK, HEMB), lambda i, c: (N // ROW_BLK + i, 0)),
            pl.BlockSpec((ROW_BLK, HEMB), lambda i, c: (i, 0)),
            pl.BlockSpec((ROW_BLK, HEMB), lambda i, c: (N // ROW_BLK + i, 0)),
            pl.BlockSpec((EMB, EMB), lambda i, c: (0, 0)),
            pl.BlockSpec((EMB,), lambda i, c: (0,)),
            pl.BlockSpec((EMB, EMB), lambda i, c: (0, 0)),
            pl.BlockSpec((EMB,), lambda i, c: (0,)),
        ],
        out_specs=[
            pl.BlockSpec((ROW_BLK, HEMB),
                         lambda i, c: (c * (N // ROW_BLK) + i, 0)),
            pl.BlockSpec((ROW_BLK, EMB), lambda i, c: (i, 0)),
        ],
        out_shape=[
            jax.ShapeDtypeStruct((2 * N, HEMB), jnp.float32),
            jax.ShapeDtypeStruct((N, EMB), jnp.float32),
        ],
    )(le_split, le_split, ego_split, ego_split, w1, b1, w2, b2)


def kernel(users, pos_items, neg_items, user_emb, item_emb, lap_idx, lap_val,
           W1_0, b1_0, W2_0, b2_0, W1_1, b1_1, W2_1, b2_1):
    ego = jnp.concatenate([user_emb, item_emb], axis=0)
    ego_split = jnp.concatenate([ego[:, :HEMB], ego[:, HEMB:]], axis=0)
    pad = NNZ_PAD - NNZ
    edges = jnp.concatenate([lap_idx, jnp.zeros((2, pad), jnp.int32)], axis=1)
    vals = jnp.concatenate([lap_val, jnp.zeros((pad,), jnp.float32)])
    outs = [ego]
    for (w1, b1, w2, b2) in ((W1_0, b1_0, W2_0, b2_0), (W1_1, b1_1, W2_1, b2_1)):
        le_split = _sc_spmm(ego_split, edges, vals)
        ego_split, norm = _dense_layer(le_split, ego_split, w1, b1, w2, b2)
        outs.append(norm)
    all_emb = jnp.concatenate(outs, axis=1)
    u_g = all_emb[:N_USER, :]
    i_g = all_emb[N_USER:, :]
    u_out = jnp.take(u_g, users - 1, axis=0, mode='wrap')
    p_out = jnp.take(i_g, pos_items - 1, axis=0, mode='wrap')
    n_out = jnp.take(i_g, neg_items - 1, axis=0, mode='wrap')
    return (u_out, p_out, n_out)
